# baseline (speedup 1.0000x reference)
import functools

import jax
import jax.numpy as jnp
from jax import lax
from jax.experimental import pallas as pl
from jax.experimental.pallas import tpu as pltpu
from jax.experimental.pallas import tpu_sc as plsc

_info = plsc.get_sparse_core_info()
_NC = _info.num_cores
_NS = _info.num_subcores
_NW = _NC * _NS

_LANES = 128


def _make_bcast(batch, rows, dtype, k):
  b_per_w = batch // _NW
  n_dma = b_per_w // k
  mesh = plsc.VectorSubcoreMesh(core_axis_name="c", subcore_axis_name="s")

  @functools.partial(
      pl.kernel,
      out_type=jax.ShapeDtypeStruct((batch * rows, _LANES), dtype),
      mesh=mesh,
      scratch_types=[
          pltpu.VMEM((k * rows, _LANES), dtype),
          pltpu.SemaphoreType.DMA,
          pltpu.SemaphoreType.DMA,
      ],
  )
  def bcast(pek_hbm, out_hbm, rep_v, sem_in, sem_out):
    cid = lax.axis_index("c")
    sid = lax.axis_index("s")
    wid = sid * _NC + cid
    base = wid * b_per_w

    pltpu.async_copy(pek_hbm, rep_v, sem_in).wait()

    outs = [
        pltpu.async_copy(
            rep_v,
            out_hbm.at[pl.ds((base + t * k) * rows, k * rows)],
            sem_out)
        for t in range(n_dma)
    ]
    for h in outs:
      h.wait()

  return bcast


def _make_bcast_exact(batch, max_len, d_model, dtype):
  b_per_w = batch // _NW
  k = 4
  while b_per_w % k:
    k //= 2
  n_dma = b_per_w // k
  mesh = plsc.VectorSubcoreMesh(core_axis_name="c", subcore_axis_name="s")

  @functools.partial(
      pl.kernel,
      out_type=jax.ShapeDtypeStruct((batch, max_len, d_model), dtype),
      mesh=mesh,
      scratch_types=[
          pltpu.VMEM((k, max_len, d_model), dtype),
          pltpu.SemaphoreType.DMA,
          pltpu.SemaphoreType.DMA,
      ],
  )
  def bcast(pe_hbm, out_hbm, rep_v, sem_in, sem_out):
    cid = lax.axis_index("c")
    sid = lax.axis_index("s")
    wid = sid * _NC + cid
    base = wid * b_per_w
    fills = [pltpu.async_copy(pe_hbm, rep_v.at[j], sem_in)
             for j in range(k)]
    for h in fills:
      h.wait()
    outs = [
        pltpu.async_copy(rep_v, out_hbm.at[pl.ds(base + t * k, k)], sem_out)
        for t in range(n_dma)
    ]
    for h in outs:
      h.wait()

  return bcast


def kernel(x, pe_weight):
  batch = x.shape[0]
  max_len, d_model = pe_weight.shape
  n = max_len * d_model
  b_per_w = batch // _NW
  if batch % _NW == 0 and n % _LANES == 0:
    rows = n // _LANES
    k = 8
    while k > 1 and (b_per_w % k or k * rows * _LANES > 131000):
      k //= 2
    pek = jnp.tile(pe_weight.reshape(rows, _LANES), (k, 1))
    out = _make_bcast(batch, rows, pe_weight.dtype, k)(pek)
    return out.reshape(batch, max_len, d_model)
  return _make_bcast_exact(batch, max_len, d_model, pe_weight.dtype)(pe_weight)


# speedup vs baseline: 1.9644x; 1.9644x over previous
import functools

import jax
import jax.numpy as jnp
from jax import lax
from jax.experimental import pallas as pl
from jax.experimental.pallas import tpu as pltpu
from jax.experimental.pallas import tpu_sc as plsc

_info = plsc.get_sparse_core_info()
_NC = _info.num_cores
_NS = _info.num_subcores
_NW = _NC * _NS

_LANES = 128


def _make_bcast(batch, rows, dtype, k):
  b_per_w = batch // _NW
  n_dma = b_per_w // k
  mesh = plsc.VectorSubcoreMesh(core_axis_name="c", subcore_axis_name="s")

  @functools.partial(
      pl.kernel,
      out_type=jax.ShapeDtypeStruct((batch, rows, _LANES), dtype),
      mesh=mesh,
      scratch_types=[
          pltpu.VMEM((k, rows, _LANES), dtype),
          pltpu.SemaphoreType.DMA,
          pltpu.SemaphoreType.DMA,
      ],
  )
  def bcast(pe_hbm, out_hbm, rep_v, sem_in, sem_out):
    cid = lax.axis_index("c")
    sid = lax.axis_index("s")
    wid = sid * _NC + cid
    base = wid * b_per_w

    fills = [pltpu.async_copy(pe_hbm, rep_v.at[j], sem_in)
             for j in range(k)]
    for h in fills:
      h.wait()

    outs = [
        pltpu.async_copy(rep_v, out_hbm.at[pl.ds(base + t * k, k)], sem_out)
        for t in range(n_dma)
    ]
    for h in outs:
      h.wait()

  return bcast


def _make_bcast_exact(batch, max_len, d_model, dtype):
  b_per_w = batch // _NW
  k = 4
  while b_per_w % k:
    k //= 2
  n_dma = b_per_w // k
  mesh = plsc.VectorSubcoreMesh(core_axis_name="c", subcore_axis_name="s")

  @functools.partial(
      pl.kernel,
      out_type=jax.ShapeDtypeStruct((batch, max_len, d_model), dtype),
      mesh=mesh,
      scratch_types=[
          pltpu.VMEM((k, max_len, d_model), dtype),
          pltpu.SemaphoreType.DMA,
          pltpu.SemaphoreType.DMA,
      ],
  )
  def bcast(pe_hbm, out_hbm, rep_v, sem_in, sem_out):
    cid = lax.axis_index("c")
    sid = lax.axis_index("s")
    wid = sid * _NC + cid
    base = wid * b_per_w
    fills = [pltpu.async_copy(pe_hbm, rep_v.at[j], sem_in)
             for j in range(k)]
    for h in fills:
      h.wait()
    outs = [
        pltpu.async_copy(rep_v, out_hbm.at[pl.ds(base + t * k, k)], sem_out)
        for t in range(n_dma)
    ]
    for h in outs:
      h.wait()

  return bcast


def kernel(x, pe_weight):
  batch = x.shape[0]
  max_len, d_model = pe_weight.shape
  n = max_len * d_model
  b_per_w = batch // _NW
  if batch % _NW == 0 and n % _LANES == 0:
    rows = n // _LANES
    k = 8
    rows_pad = (rows + 7) // 8 * 8
    while k > 1 and (b_per_w % k or k * rows_pad * _LANES > 131000):
      k //= 2
    pe2 = pe_weight.reshape(rows, _LANES)
    out = _make_bcast(batch, rows, pe_weight.dtype, k)(pe2)
    return out.reshape(batch, max_len, d_model)
  return _make_bcast_exact(batch, max_len, d_model, pe_weight.dtype)(pe_weight)
